# baseline (speedup 1.0000x reference)
import jax
import jax.numpy as jnp
from jax import lax
from jax.experimental import pallas as pl
from jax.experimental.pallas import tpu as pltpu
from jax.experimental.pallas import tpu_sc as plsc

_B, _C, _L, _N = 16, 512, 4096, 1024
_GAMMA = 1.0
_CW = 16
_HC = 8
_LANES = 16
_NG = _N // _LANES
_NW = 32
_WBLK = _N * _CW


def _sampler_body(inp, pt, off, out, pt_v, off_v, i0_v, w1_v, inb0, inb1,
                  outb, sem0, sem1):
    cid = lax.axis_index("c")
    sid = lax.axis_index("s")
    wid = cid * 16 + sid
    c0 = wid * _CW

    def idx_body(j, _):
        sl = pl.ds(j * _LANES, _LANES)
        loc = pt_v[sl] + _GAMMA * off_v[sl]
        loc = jnp.minimum(jnp.maximum(loc, 0.0), float(_L - 1))
        i0 = loc.astype(jnp.int32)
        i0_v[sl] = i0
        w1_v[sl] = loc - i0.astype(jnp.float32)
        return 0

    def compute_half(buf, h):
        def grp_body(g, _):
            n_base = g * _LANES
            sl = pl.ds(n_base, _LANES)
            i0 = i0_v[sl]
            w1 = w1_v[sl]
            i1 = jnp.minimum(i0 + 1, _L - 1)
            w0 = 1.0 - w1
            o_base = n_base * _CW + h * _HC
            o_idx = o_base + lax.iota(jnp.int32, _LANES) * _CW
            for c in range(_HC):
                c_idx = jnp.full((_LANES,), c, jnp.int32)
                v0 = plsc.load_gather(buf, [c_idx, i0])
                v1 = plsc.load_gather(buf, [c_idx, i1])
                r = w0 * v0 + w1 * v1
                plsc.store_scatter(outb, [o_idx + c], r)
            return 0

        lax.fori_loop(0, _NG, grp_body, 0)

    def in_slab(b, h):
        return inp.at[b, pl.ds(c0 + h * _HC, _HC)]

    pltpu.async_copy(in_slab(0, 0), inb0, sem0)

    def per_batch(b, _):
        pltpu.sync_copy(pt.at[pl.ds(b * _N, _N)], pt_v)
        pltpu.sync_copy(off.at[pl.ds(b * _N, _N)], off_v)
        lax.fori_loop(0, _NG, idx_body, 0)

        pltpu.async_copy(in_slab(b, 1), inb1, sem1)
        pltpu.make_async_copy(in_slab(b, 0), inb0, sem0).wait()
        compute_half(inb0, 0)

        @pl.when(b + 1 < _B)
        def _():
            pltpu.async_copy(in_slab(b + 1, 0), inb0, sem0)

        pltpu.make_async_copy(in_slab(b, 1), inb1, sem1).wait()
        compute_half(inb1, 1)

        pltpu.sync_copy(outb, out.at[pl.ds((wid * _B + b) * _WBLK, _WBLK)])
        return 0

    lax.fori_loop(0, _B, per_batch, 0)


def kernel(input, point, offset):
    pt = point.reshape(_B * _N)
    off = offset.reshape(_B * _N)
    mesh = plsc.VectorSubcoreMesh(core_axis_name="c", subcore_axis_name="s")
    f = pl.kernel(
        _sampler_body,
        out_type=jax.ShapeDtypeStruct((_NW * _B * _WBLK,), jnp.float32),
        mesh=mesh,
        scratch_types=[
            pltpu.VMEM((_N,), jnp.float32),
            pltpu.VMEM((_N,), jnp.float32),
            pltpu.VMEM((_N,), jnp.int32),
            pltpu.VMEM((_N,), jnp.float32),
            pltpu.VMEM((_HC, _L), jnp.float32),
            pltpu.VMEM((_HC, _L), jnp.float32),
            pltpu.VMEM((_WBLK,), jnp.float32),
            pltpu.SemaphoreType.DMA,
            pltpu.SemaphoreType.DMA,
        ],
        compiler_params=pltpu.CompilerParams(needs_layout_passes=False),
    )
    staged = f(input, pt, off)
    out = staged.reshape(_NW, _B, _N, _CW).transpose(1, 2, 0, 3)
    return out.reshape(_B, _N, _C)


# speedup vs baseline: 1.3170x; 1.3170x over previous
import jax
import jax.numpy as jnp
from jax import lax
from jax.experimental import pallas as pl
from jax.experimental.pallas import tpu as pltpu
from jax.experimental.pallas import tpu_sc as plsc

_B, _C, _L, _N = 16, 512, 4096, 1024
_GAMMA = 1.0
_CW = 16
_HC = 4
_NQ = _CW // _HC
_LANES = 16
_NG = _N // _LANES
_WBLK = _N * _CW
_NSUB = 16
_ROWS = _N // _NSUB
_CCORE = _NSUB * _CW


def _sampler_body(inp, loc_in, out, loc_v, i0_v, w1_v, inb0, inb1,
                  outb, tmp, mrg, shm, sem0, sem1):
    cid = lax.axis_index("c")
    sid = lax.axis_index("s")
    wid = cid * _NSUB + sid
    c0 = wid * _CW

    def idx_body(j, _):
        sl = pl.ds(j * _LANES, _LANES)
        loc = loc_v[sl]
        i0 = loc.astype(jnp.int32)
        i0_v[sl] = i0
        w1_v[sl] = loc - i0.astype(jnp.float32)
        return 0

    def compute_quarter(buf, q):
        def grp_body(g, _):
            n_base = g * _LANES
            sl = pl.ds(n_base, _LANES)
            i0 = i0_v[sl]
            w1 = w1_v[sl]
            i1 = jnp.minimum(i0 + 1, _L - 1)
            w0 = 1.0 - w1
            o_idx = (n_base * _CW + q * _HC) + lax.iota(jnp.int32, _LANES) * _CW
            for c in range(_HC):
                c_idx = jnp.full((_LANES,), c, jnp.int32)
                v0 = plsc.load_gather(buf, [c_idx, i0])
                v1 = plsc.load_gather(buf, [c_idx, i1])
                r = w0 * v0 + w1 * v1
                plsc.store_scatter(outb, [o_idx + c], r)
            return 0

        lax.fori_loop(0, _NG, grp_body, 0)

    def in_slab(b, q):
        return inp.at[b, pl.ds(c0 + q * _HC, _HC)]

    bufs = (inb0, inb1)
    sems = (sem0, sem1)

    pltpu.async_copy(in_slab(0, 0), inb0, sem0)

    def per_batch(b, _):
        pltpu.sync_copy(loc_in.at[pl.ds(b * _N, _N)], loc_v)
        lax.fori_loop(0, _NG, idx_body, 0)

        for q in range(_NQ):
            if q + 1 < _NQ:
                pltpu.async_copy(in_slab(b, q + 1),
                                 bufs[(q + 1) % 2], sems[(q + 1) % 2])
            else:
                @pl.when(b + 1 < _B)
                def _():
                    pltpu.async_copy(in_slab(b + 1, 0), bufs[0], sems[0])

            pltpu.make_async_copy(in_slab(b, q), bufs[q % 2], sems[q % 2]).wait()
            compute_quarter(bufs[q % 2], q)

        pltpu.sync_copy(outb, shm.at[pl.ds(sid * _WBLK, _WBLK)])
        plsc.subcore_barrier()
        for j in range(_NSUB):
            pltpu.sync_copy(
                shm.at[pl.ds(j * _WBLK + sid * _ROWS * _CW, _ROWS * _CW)],
                tmp.at[pl.ds(j * _ROWS * _CW, _ROWS * _CW)],
            )
        def row_body(n, _):
            for j in range(_NSUB):
                v = tmp[pl.ds(j * _ROWS * _CW + n * _CW, _LANES)]
                plsc.store_scatter(
                    mrg, [jnp.full((_LANES,), n, jnp.int32),
                          j * _CW + lax.iota(jnp.int32, _LANES)], v)
            return 0

        lax.fori_loop(0, _ROWS, row_body, 0)
        pltpu.sync_copy(
            mrg,
            out.at[b, pl.ds(sid * _ROWS, _ROWS), pl.ds(cid * _CCORE, _CCORE)],
        )
        plsc.subcore_barrier()
        return 0

    lax.fori_loop(0, _B, per_batch, 0)


def kernel(input, point, offset):
    loc = jnp.clip(point[:, :, 0] + _GAMMA * offset[:, :, 0], 0.0,
                   float(_L - 1)).reshape(_B * _N)
    mesh = plsc.VectorSubcoreMesh(core_axis_name="c", subcore_axis_name="s")
    f = pl.kernel(
        _sampler_body,
        out_type=jax.ShapeDtypeStruct((_B, _N, _C), jnp.float32),
        mesh=mesh,
        scratch_types=[
            pltpu.VMEM((_N,), jnp.float32),
            pltpu.VMEM((_N,), jnp.int32),
            pltpu.VMEM((_N,), jnp.float32),
            pltpu.VMEM((_HC, _L), jnp.float32),
            pltpu.VMEM((_HC, _L), jnp.float32),
            pltpu.VMEM((_WBLK,), jnp.float32),
            pltpu.VMEM((_WBLK,), jnp.float32),
            pltpu.VMEM((_ROWS, _CCORE), jnp.float32),
            pltpu.VMEM_SHARED((_NSUB * _WBLK,), jnp.float32),
            pltpu.SemaphoreType.DMA,
            pltpu.SemaphoreType.DMA,
        ],
        compiler_params=pltpu.CompilerParams(needs_layout_passes=False),
    )
    return f(input, loc)


# speedup vs baseline: 1.4164x; 1.0754x over previous
import jax
import jax.numpy as jnp
from jax import lax
from jax.experimental import pallas as pl
from jax.experimental.pallas import tpu as pltpu
from jax.experimental.pallas import tpu_sc as plsc

_B, _C, _L, _N = 16, 512, 4096, 1024
_GAMMA = 1.0
_CW = 16
_HC = 4
_NQ = _CW // _HC
_LANES = 16
_NG = _N // _LANES
_WBLK = _N * _CW
_NSUB = 16
_ROWS = _N // _NSUB
_CCORE = _NSUB * _CW


def _sampler_body(inp, loc_in, out, loc_v, i0_v, w1_v, inb0, inb1,
                  outb, tmp, mrg, shm, sem0, sem1):
    cid = lax.axis_index("c")
    sid = lax.axis_index("s")
    wid = cid * _NSUB + sid
    c0 = wid * _CW

    def idx_body(j, _):
        sl = pl.ds(j * _LANES, _LANES)
        loc = loc_v[sl]
        i0 = loc.astype(jnp.int32)
        i0_v[sl] = i0
        w1_v[sl] = loc - i0.astype(jnp.float32)
        return 0

    def compute_quarter(buf, q):
        def grp_body(g, _):
            n_base = g * _LANES
            sl = pl.ds(n_base, _LANES)
            i0 = i0_v[sl]
            w1 = w1_v[sl]
            i1 = jnp.minimum(i0 + 1, _L - 1)
            w0 = 1.0 - w1
            s_idx = jnp.full((_LANES,), g // 4, jnp.int32)
            p_base = ((n_base % 64) + lax.iota(jnp.int32, _LANES)) * _CW + q * _HC
            for c in range(_HC):
                c_idx = jnp.full((_LANES,), c, jnp.int32)
                v0 = plsc.load_gather(buf, [c_idx, i0])
                v1 = plsc.load_gather(buf, [c_idx, i1])
                r = w0 * v0 + w1 * v1
                p = p_base + c
                plsc.store_scatter(outb, [s_idx, p >> 7, p & 127], r)
            return 0

        lax.fori_loop(0, _NG, grp_body, 0)

    def in_slab(b, q):
        return inp.at[b, pl.ds(c0 + q * _HC, _HC)]

    bufs = (inb0, inb1)
    sems = (sem0, sem1)

    pltpu.async_copy(in_slab(0, 0), inb0, sem0)

    def per_batch(b, _):
        pltpu.sync_copy(loc_in.at[pl.ds(b * _N, _N)], loc_v)
        lax.fori_loop(0, _NG, idx_body, 0)

        for q in range(_NQ):
            if q + 1 < _NQ:
                pltpu.async_copy(in_slab(b, q + 1),
                                 bufs[(q + 1) % 2], sems[(q + 1) % 2])
            else:
                @pl.when(b + 1 < _B)
                def _():
                    pltpu.async_copy(in_slab(b + 1, 0), bufs[0], sems[0])

            pltpu.make_async_copy(in_slab(b, q), bufs[q % 2], sems[q % 2]).wait()
            compute_quarter(bufs[q % 2], q)

        pltpu.sync_copy(outb, shm.at[sid])
        plsc.subcore_barrier()
        pltpu.sync_copy(shm.at[:, sid], tmp)
        def row_body(n, _):
            po = (n % 8) * _CW
            ph = n // 8
            for j in range(_NSUB):
                v = tmp[j, ph, pl.ds(po, _LANES)]
                plsc.store_scatter(
                    mrg, [jnp.full((_LANES,), n, jnp.int32),
                          j * _CW + lax.iota(jnp.int32, _LANES)], v)
            return 0

        lax.fori_loop(0, _ROWS, row_body, 0)
        pltpu.sync_copy(
            mrg,
            out.at[b, pl.ds(sid * _ROWS, _ROWS), pl.ds(cid * _CCORE, _CCORE)],
        )
        plsc.subcore_barrier()
        return 0

    lax.fori_loop(0, _B, per_batch, 0)


def kernel(input, point, offset):
    loc = jnp.clip(point[:, :, 0] + _GAMMA * offset[:, :, 0], 0.0,
                   float(_L - 1)).reshape(_B * _N)
    mesh = plsc.VectorSubcoreMesh(core_axis_name="c", subcore_axis_name="s")
    f = pl.kernel(
        _sampler_body,
        out_type=jax.ShapeDtypeStruct((_B, _N, _C), jnp.float32),
        mesh=mesh,
        scratch_types=[
            pltpu.VMEM((_N,), jnp.float32),
            pltpu.VMEM((_N,), jnp.int32),
            pltpu.VMEM((_N,), jnp.float32),
            pltpu.VMEM((_HC, _L), jnp.float32),
            pltpu.VMEM((_HC, _L), jnp.float32),
            pltpu.VMEM((_NSUB, 8, 128), jnp.float32),
            pltpu.VMEM((_NSUB, 8, 128), jnp.float32),
            pltpu.VMEM((_ROWS, _CCORE), jnp.float32),
            pltpu.VMEM_SHARED((_NSUB, _NSUB, 8, 128), jnp.float32),
            pltpu.SemaphoreType.DMA,
            pltpu.SemaphoreType.DMA,
        ],
        compiler_params=pltpu.CompilerParams(needs_layout_passes=False),
    )
    return f(input, loc)


# speedup vs baseline: 1.9929x; 1.4071x over previous
import jax
import jax.numpy as jnp
from jax import lax
from jax.experimental import pallas as pl
from jax.experimental.pallas import tpu as pltpu
from jax.experimental.pallas import tpu_sc as plsc

_B, _C, _L, _N = 16, 512, 4096, 1024
_GAMMA = 1.0
_CW = 16
_HC = 4
_NQ = _CW // _HC
_LANES = 16
_NG = _N // _LANES
_WBLK = _N * _CW
_NSUB = 16
_ROWS = _N // _NSUB
_CCORE = _NSUB * _CW


def _sampler_body(inp, loc_in, out, loc_v, i0_v, w1_v, inb0, inb1,
                  outb, tmp, mrg, shm, sem0, sem1):
    cid = lax.axis_index("c")
    sid = lax.axis_index("s")
    wid = cid * _NSUB + sid
    c0 = wid * _CW

    def run_idx_loop():
        @plsc.parallel_loop(0, _NG, unroll=2)
        def idx_body(j):
            sl = pl.ds(j * _LANES, _LANES)
            loc = loc_v[sl]
            i0 = loc.astype(jnp.int32)
            i0_v[sl] = i0
            w1_v[sl] = loc - i0.astype(jnp.float32)

    def compute_quarter(buf, q):
        @plsc.parallel_loop(0, _NG, unroll=4)
        def grp_body(g):
            n_base = g * _LANES
            sl = pl.ds(n_base, _LANES)
            i0 = i0_v[sl]
            w1 = w1_v[sl]
            i1 = jnp.minimum(i0 + 1, _L - 1)
            w0 = 1.0 - w1
            s_idx = jnp.full((_LANES,), g // 4, jnp.int32)
            p_base = ((n_base % 64) + lax.iota(jnp.int32, _LANES)) * _CW + q * _HC
            for c in range(_HC):
                c_idx = jnp.full((_LANES,), c, jnp.int32)
                v0 = plsc.load_gather(buf, [c_idx, i0])
                v1 = plsc.load_gather(buf, [c_idx, i1])
                r = w0 * v0 + w1 * v1
                p = p_base + c
                plsc.store_scatter(outb, [s_idx, p >> 7, p & 127], r)

    def in_slab(b, q):
        return inp.at[b, pl.ds(c0 + q * _HC, _HC)]

    bufs = (inb0, inb1)
    sems = (sem0, sem1)

    pltpu.async_copy(in_slab(0, 0), inb0, sem0)

    def per_batch(b, _):
        pltpu.sync_copy(loc_in.at[pl.ds(b * _N, _N)], loc_v)
        run_idx_loop()

        for q in range(_NQ):
            if q + 1 < _NQ:
                pltpu.async_copy(in_slab(b, q + 1),
                                 bufs[(q + 1) % 2], sems[(q + 1) % 2])
            else:
                @pl.when(b + 1 < _B)
                def _():
                    pltpu.async_copy(in_slab(b + 1, 0), bufs[0], sems[0])

            pltpu.make_async_copy(in_slab(b, q), bufs[q % 2], sems[q % 2]).wait()
            compute_quarter(bufs[q % 2], q)

        pltpu.sync_copy(outb, shm.at[sid])
        plsc.subcore_barrier()
        pltpu.sync_copy(shm.at[:, sid], tmp)
        @plsc.parallel_loop(0, _ROWS, unroll=2)
        def row_body(n):
            po = (n % 8) * _CW
            ph = n // 8
            for j in range(_NSUB):
                v = tmp[j, ph, pl.ds(po, _LANES)]
                plsc.store_scatter(
                    mrg, [jnp.full((_LANES,), n, jnp.int32),
                          j * _CW + lax.iota(jnp.int32, _LANES)], v)
        pltpu.sync_copy(
            mrg,
            out.at[b, pl.ds(sid * _ROWS, _ROWS), pl.ds(cid * _CCORE, _CCORE)],
        )
        plsc.subcore_barrier()
        return 0

    lax.fori_loop(0, _B, per_batch, 0)


def kernel(input, point, offset):
    loc = jnp.clip(point[:, :, 0] + _GAMMA * offset[:, :, 0], 0.0,
                   float(_L - 1)).reshape(_B * _N)
    mesh = plsc.VectorSubcoreMesh(core_axis_name="c", subcore_axis_name="s")
    f = pl.kernel(
        _sampler_body,
        out_type=jax.ShapeDtypeStruct((_B, _N, _C), jnp.float32),
        mesh=mesh,
        scratch_types=[
            pltpu.VMEM((_N,), jnp.float32),
            pltpu.VMEM((_N,), jnp.int32),
            pltpu.VMEM((_N,), jnp.float32),
            pltpu.VMEM((_HC, _L), jnp.float32),
            pltpu.VMEM((_HC, _L), jnp.float32),
            pltpu.VMEM((_NSUB, 8, 128), jnp.float32),
            pltpu.VMEM((_NSUB, 8, 128), jnp.float32),
            pltpu.VMEM((_ROWS, _CCORE), jnp.float32),
            pltpu.VMEM_SHARED((_NSUB, _NSUB, 8, 128), jnp.float32),
            pltpu.SemaphoreType.DMA,
            pltpu.SemaphoreType.DMA,
        ],
        compiler_params=pltpu.CompilerParams(needs_layout_passes=False),
    )
    return f(input, loc)


# speedup vs baseline: 1.9993x; 1.0032x over previous
import jax
import jax.numpy as jnp
from jax import lax
from jax.experimental import pallas as pl
from jax.experimental.pallas import tpu as pltpu
from jax.experimental.pallas import tpu_sc as plsc

_B, _C, _L, _N = 16, 512, 4096, 1024
_GAMMA = 1.0
_CW = 16
_HC = 4
_NQ = _CW // _HC
_LANES = 16
_NG = _N // _LANES
_WBLK = _N * _CW
_NSUB = 16
_ROWS = _N // _NSUB
_CCORE = _NSUB * _CW


def _sampler_body(inp, loc_in, out, loc_v, i0_v, w1_v, inb0, inb1,
                  outb, tmp, mrg, shm, sem0, sem1):
    cid = lax.axis_index("c")
    sid = lax.axis_index("s")
    wid = cid * _NSUB + sid
    c0 = wid * _CW

    def run_idx_loop():
        @plsc.parallel_loop(0, _NG, unroll=2)
        def idx_body(j):
            sl = pl.ds(j * _LANES, _LANES)
            loc = loc_v[sl]
            i0 = loc.astype(jnp.int32)
            i0_v[sl] = i0
            w1_v[sl] = loc - i0.astype(jnp.float32)

    def compute_quarter(buf, q):
        @plsc.parallel_loop(0, _NG, unroll=8)
        def grp_body(g):
            n_base = g * _LANES
            sl = pl.ds(n_base, _LANES)
            i0 = i0_v[sl]
            w1 = w1_v[sl]
            i1 = jnp.minimum(i0 + 1, _L - 1)
            w0 = 1.0 - w1
            s_idx = jnp.full((_LANES,), g // 4, jnp.int32)
            p_base = ((n_base % 64) + lax.iota(jnp.int32, _LANES)) * _CW + q * _HC
            for c in range(_HC):
                c_idx = jnp.full((_LANES,), c, jnp.int32)
                v0 = plsc.load_gather(buf, [c_idx, i0])
                v1 = plsc.load_gather(buf, [c_idx, i1])
                r = w0 * v0 + w1 * v1
                p = p_base + c
                plsc.store_scatter(outb, [s_idx, p >> 7, p & 127], r)

    def in_slab(b, q):
        return inp.at[b, pl.ds(c0 + q * _HC, _HC)]

    bufs = (inb0, inb1)
    sems = (sem0, sem1)

    pltpu.async_copy(in_slab(0, 0), inb0, sem0)

    def per_batch(b, _):
        pltpu.sync_copy(loc_in.at[pl.ds(b * _N, _N)], loc_v)
        run_idx_loop()

        for q in range(_NQ):
            if q + 1 < _NQ:
                pltpu.async_copy(in_slab(b, q + 1),
                                 bufs[(q + 1) % 2], sems[(q + 1) % 2])
            else:
                @pl.when(b + 1 < _B)
                def _():
                    pltpu.async_copy(in_slab(b + 1, 0), bufs[0], sems[0])

            pltpu.make_async_copy(in_slab(b, q), bufs[q % 2], sems[q % 2]).wait()
            compute_quarter(bufs[q % 2], q)

        pltpu.sync_copy(outb, shm.at[sid])
        plsc.subcore_barrier()
        pltpu.sync_copy(shm.at[:, sid], tmp)
        @plsc.parallel_loop(0, _ROWS, unroll=2)
        def row_body(n):
            po = (n % 8) * _CW
            ph = n // 8
            for j in range(_NSUB):
                v = tmp[j, ph, pl.ds(po, _LANES)]
                plsc.store_scatter(
                    mrg, [jnp.full((_LANES,), n, jnp.int32),
                          j * _CW + lax.iota(jnp.int32, _LANES)], v)
        pltpu.sync_copy(
            mrg,
            out.at[b, pl.ds(sid * _ROWS, _ROWS), pl.ds(cid * _CCORE, _CCORE)],
        )
        plsc.subcore_barrier()
        return 0

    lax.fori_loop(0, _B, per_batch, 0)


def kernel(input, point, offset):
    loc = jnp.clip(point[:, :, 0] + _GAMMA * offset[:, :, 0], 0.0,
                   float(_L - 1)).reshape(_B * _N)
    mesh = plsc.VectorSubcoreMesh(core_axis_name="c", subcore_axis_name="s")
    f = pl.kernel(
        _sampler_body,
        out_type=jax.ShapeDtypeStruct((_B, _N, _C), jnp.float32),
        mesh=mesh,
        scratch_types=[
            pltpu.VMEM((_N,), jnp.float32),
            pltpu.VMEM((_N,), jnp.int32),
            pltpu.VMEM((_N,), jnp.float32),
            pltpu.VMEM((_HC, _L), jnp.float32),
            pltpu.VMEM((_HC, _L), jnp.float32),
            pltpu.VMEM((_NSUB, 8, 128), jnp.float32),
            pltpu.VMEM((_NSUB, 8, 128), jnp.float32),
            pltpu.VMEM((_ROWS, _CCORE), jnp.float32),
            pltpu.VMEM_SHARED((_NSUB, _NSUB, 8, 128), jnp.float32),
            pltpu.SemaphoreType.DMA,
            pltpu.SemaphoreType.DMA,
        ],
        compiler_params=pltpu.CompilerParams(needs_layout_passes=False),
    )
    return f(input, loc)


# speedup vs baseline: 2.0047x; 1.0027x over previous
import jax
import jax.numpy as jnp
from jax import lax
from jax.experimental import pallas as pl
from jax.experimental.pallas import tpu as pltpu
from jax.experimental.pallas import tpu_sc as plsc

_B, _C, _L, _N = 16, 512, 4096, 1024
_GAMMA = 1.0
_CW = 16
_HC = 4
_NQ = _CW // _HC
_LANES = 16
_NG = _N // _LANES
_WBLK = _N * _CW
_NSUB = 16
_ROWS = _N // _NSUB
_CCORE = _NSUB * _CW


def _sampler_body(inp, loc_in, out, loc_v, i0_v, w1_v, inb0, inb1,
                  outb, tmp, mrg, shm, sem0, sem1):
    cid = lax.axis_index("c")
    sid = lax.axis_index("s")
    wid = cid * _NSUB + sid
    c0 = wid * _CW

    def run_idx_loop():
        @plsc.parallel_loop(0, _NG, unroll=2)
        def idx_body(j):
            sl = pl.ds(j * _LANES, _LANES)
            loc = loc_v[sl]
            i0 = loc.astype(jnp.int32)
            i0_v[sl] = i0
            w1_v[sl] = loc - i0.astype(jnp.float32)

    def compute_quarter(buf, q):
        @plsc.parallel_loop(0, _NG, unroll=8)
        def grp_body(g):
            n_base = g * _LANES
            sl = pl.ds(n_base, _LANES)
            i0 = i0_v[sl]
            w1 = w1_v[sl]
            i1 = jnp.minimum(i0 + 1, _L - 1)
            w0 = 1.0 - w1
            s_idx = jnp.full((_LANES,), g // 4, jnp.int32)
            p_base = ((n_base % 64) + lax.iota(jnp.int32, _LANES)) * _CW + q * _HC
            for c in range(_HC):
                c_idx = jnp.full((_LANES,), c, jnp.int32)
                v0 = plsc.load_gather(buf, [c_idx, i0])
                v1 = plsc.load_gather(buf, [c_idx, i1])
                r = w0 * v0 + w1 * v1
                p = p_base + c
                plsc.store_scatter(outb, [s_idx, p >> 7, p & 127], r)

    def in_slab(b, q):
        return inp.at[b, pl.ds(c0 + q * _HC, _HC)]

    bufs = (inb0, inb1)
    sems = (sem0, sem1)

    pltpu.async_copy(in_slab(0, 0), inb0, sem0)

    def per_batch(b, _):
        with jax.named_scope("idx_phase"):
            pltpu.sync_copy(loc_in.at[pl.ds(b * _N, _N)], loc_v)
            run_idx_loop()

        for q in range(_NQ):
            if q + 1 < _NQ:
                pltpu.async_copy(in_slab(b, q + 1),
                                 bufs[(q + 1) % 2], sems[(q + 1) % 2])
            else:
                @pl.when(b + 1 < _B)
                def _():
                    pltpu.async_copy(in_slab(b + 1, 0), bufs[0], sems[0])

            with jax.named_scope("in_wait"):
                pltpu.make_async_copy(in_slab(b, q), bufs[q % 2],
                                      sems[q % 2]).wait()
            with jax.named_scope("gather"):
                compute_quarter(bufs[q % 2], q)

        with jax.named_scope("mrg_put"):
            pltpu.sync_copy(outb, shm.at[sid])
        with jax.named_scope("mrg_bar1"):
            plsc.subcore_barrier()
        with jax.named_scope("mrg_get"):
            pltpu.sync_copy(shm.at[:, sid], tmp)
        with jax.named_scope("mrg_ilv"):
            @plsc.parallel_loop(0, _ROWS, unroll=2)
            def row_body(n):
                po = (n % 8) * _CW
                ph = n // 8
                for j in range(_NSUB):
                    v = tmp[j, ph, pl.ds(po, _LANES)]
                    plsc.store_scatter(
                        mrg, [jnp.full((_LANES,), n, jnp.int32),
                              j * _CW + lax.iota(jnp.int32, _LANES)], v)
        with jax.named_scope("out_dma"):
            pltpu.sync_copy(
                mrg,
                out.at[b, pl.ds(sid * _ROWS, _ROWS),
                       pl.ds(cid * _CCORE, _CCORE)],
            )
        with jax.named_scope("mrg_bar2"):
            plsc.subcore_barrier()
        return 0

    lax.fori_loop(0, _B, per_batch, 0)


def kernel(input, point, offset):
    loc = jnp.clip(point[:, :, 0] + _GAMMA * offset[:, :, 0], 0.0,
                   float(_L - 1)).reshape(_B * _N)
    mesh = plsc.VectorSubcoreMesh(core_axis_name="c", subcore_axis_name="s")
    f = pl.kernel(
        _sampler_body,
        out_type=jax.ShapeDtypeStruct((_B, _N, _C), jnp.float32),
        mesh=mesh,
        scratch_types=[
            pltpu.VMEM((_N,), jnp.float32),
            pltpu.VMEM((_N,), jnp.int32),
            pltpu.VMEM((_N,), jnp.float32),
            pltpu.VMEM((_HC, _L), jnp.float32),
            pltpu.VMEM((_HC, _L), jnp.float32),
            pltpu.VMEM((_NSUB, 8, 128), jnp.float32),
            pltpu.VMEM((_NSUB, 8, 128), jnp.float32),
            pltpu.VMEM((_ROWS, _CCORE), jnp.float32),
            pltpu.VMEM_SHARED((_NSUB, _NSUB, 8, 128), jnp.float32),
            pltpu.SemaphoreType.DMA,
            pltpu.SemaphoreType.DMA,
        ],
        compiler_params=pltpu.CompilerParams(needs_layout_passes=False),
    )
    return f(input, loc)
